# baseline (speedup 1.0000x reference)
import functools

import jax
import jax.numpy as jnp
from jax.experimental import pallas as pl


def _body(x_ref, w_ref, b_ref, g_ref, o_ref):
    xb = x_ref[...]
    pooled = jnp.mean(xb, axis=2)
    w = w_ref[...]
    cols = [
        jnp.sum(pooled * w[k][None, :], axis=1, keepdims=True)
        for k in range(3)
    ]
    logits = jnp.concatenate(cols, axis=1) + b_ref[...]
    p = jax.nn.softmax(logits, axis=1)
    y = jnp.log(p + 1e-12) + g_ref[...]
    y0, y1, y2 = y[:, 0:1], y[:, 1:2], y[:, 2:3]
    i01 = jnp.where(y1 > y0, 1, 0)
    m01 = jnp.maximum(y0, y1)
    idx = jnp.where(y2 > m01, 2, i01)
    o_ref[...] = idx.astype(jnp.int32)


@functools.partial(jax.jit, static_argnames=())
def kernel(x, W, b):
    N, C, H, Wd = x.shape
    S = H * Wd
    K = W.shape[0]
    x3 = x.reshape(N, C, S)
    b2 = b.reshape(1, K)
    g = jax.random.gumbel(jax.random.key(42), (N, K), jnp.float32)

    bn = 16
    out = pl.pallas_call(
        _body,
        grid=(N // bn,),
        in_specs=[
            pl.BlockSpec((bn, C, S), lambda i: (i, 0, 0)),
            pl.BlockSpec((K, C), lambda i: (0, 0)),
            pl.BlockSpec((1, K), lambda i: (0, 0)),
            pl.BlockSpec((bn, K), lambda i: (i, 0)),
        ],
        out_specs=pl.BlockSpec((bn, 1), lambda i: (i, 0)),
        out_shape=jax.ShapeDtypeStruct((N, 1), jnp.int32),
    )(x3, W, b2, g)
    return out.reshape(N)


# speedup vs baseline: 1.2806x; 1.2806x over previous
import jax
import jax.numpy as jnp
from jax.experimental import pallas as pl


def _body(x_ref, wt_ref, b_ref, g_ref, o_ref):
    xb = x_ref[...]
    pooled = jnp.sum(xb, axis=2, keepdims=True) / x_ref.shape[2]
    prod = pooled * wt_ref[...][None, :, :]
    logits = jnp.sum(prod, axis=1) + b_ref[...]
    p = jax.nn.softmax(logits, axis=1)
    y = jnp.log(p + 1e-12) + g_ref[...]
    y0, y1, y2 = y[:, 0:1], y[:, 1:2], y[:, 2:3]
    i01 = jnp.where(y1 > y0, 1, 0)
    m01 = jnp.maximum(y0, y1)
    idx = jnp.where(y2 > m01, 2, i01)
    o_ref[...] = idx.astype(jnp.int32)


def kernel(x, W, b):
    N, C, H, Wd = x.shape
    S = H * Wd
    K = W.shape[0]
    x3 = x.reshape(N, C, S)
    Wt = W.T
    b2 = b.reshape(1, K)
    g = jax.random.gumbel(jax.random.key(42), (N, K), jnp.float32)

    bn = 16
    out = pl.pallas_call(
        _body,
        grid=(N // bn,),
        in_specs=[
            pl.BlockSpec((bn, C, S), lambda i: (i, 0, 0)),
            pl.BlockSpec((C, K), lambda i: (0, 0)),
            pl.BlockSpec((1, K), lambda i: (0, 0)),
            pl.BlockSpec((bn, K), lambda i: (i, 0)),
        ],
        out_specs=pl.BlockSpec((bn, 1), lambda i: (i, 0)),
        out_shape=jax.ShapeDtypeStruct((N, 1), jnp.int32),
    )(x3, Wt, b2, g)
    return out.reshape(N)
